# baseline (speedup 1.0000x reference)
import jax
import jax.numpy as jnp
from jax.experimental import pallas as pl

_BM = 512


def _router_block(x_ref, w1_ref, b1_ref, w2_ref, b2_ref, o_ref):
    x = x_ref[...].astype(jnp.bfloat16)
    h = jnp.dot(x, w1_ref[...], preferred_element_type=jnp.float32)
    h = jnp.maximum(h + b1_ref[...], 0.0)
    logits = jnp.dot(h, w2_ref[...], preferred_element_type=jnp.float32)
    logits = logits + b2_ref[...]
    m = jnp.max(logits, axis=1, keepdims=True)
    e = jnp.exp(logits - m)
    o_ref[...] = e / jnp.sum(e, axis=1, keepdims=True)


def kernel(prnet_features, W1, b1, W2, b2):
    n, d = prnet_features.shape
    hidden = W1.shape[1]
    ne = W2.shape[1]
    return pl.pallas_call(
        _router_block,
        grid=(n // _BM,),
        in_specs=[
            pl.BlockSpec((_BM, d), lambda i: (i, 0)),
            pl.BlockSpec((d, hidden), lambda i: (0, 0)),
            pl.BlockSpec((1, hidden), lambda i: (0, 0)),
            pl.BlockSpec((hidden, ne), lambda i: (0, 0)),
            pl.BlockSpec((1, ne), lambda i: (0, 0)),
        ],
        out_specs=pl.BlockSpec((_BM, ne), lambda i: (i, 0)),
        out_shape=jax.ShapeDtypeStruct((n, ne), jnp.float32),
    )(prnet_features, W1.astype(jnp.bfloat16), b1.reshape(1, hidden),
      W2, b2.reshape(1, ne))


# speedup vs baseline: 1.0369x; 1.0369x over previous
import jax
import jax.numpy as jnp
from jax.experimental import pallas as pl

_BM = 1024


def _router_block(x_ref, w1_ref, b1_ref, w2_ref, b2_ref, o_ref):
    x = x_ref[...].astype(jnp.bfloat16)
    h = jnp.dot(x, w1_ref[...], preferred_element_type=jnp.float32)
    h = jnp.maximum(h + b1_ref[...], 0.0)
    logits = jnp.dot(h, w2_ref[...], preferred_element_type=jnp.float32)
    logits = logits + b2_ref[...]
    m = jnp.max(logits, axis=1, keepdims=True)
    e = jnp.exp(logits - m)
    o_ref[...] = e / jnp.sum(e, axis=1, keepdims=True)


def kernel(prnet_features, W1, b1, W2, b2):
    n, d = prnet_features.shape
    hidden = W1.shape[1]
    ne = W2.shape[1]
    return pl.pallas_call(
        _router_block,
        grid=(n // _BM,),
        in_specs=[
            pl.BlockSpec((_BM, d), lambda i: (i, 0)),
            pl.BlockSpec((d, hidden), lambda i: (0, 0)),
            pl.BlockSpec((1, hidden), lambda i: (0, 0)),
            pl.BlockSpec((hidden, ne), lambda i: (0, 0)),
            pl.BlockSpec((1, ne), lambda i: (0, 0)),
        ],
        out_specs=pl.BlockSpec((_BM, ne), lambda i: (i, 0)),
        out_shape=jax.ShapeDtypeStruct((n, ne), jnp.float32),
    )(prnet_features, W1.astype(jnp.bfloat16), b1.reshape(1, hidden),
      W2, b2.reshape(1, ne))
